# baseline (speedup 1.0000x reference)
import jax
import jax.numpy as jnp
from jax import lax
from jax.experimental import pallas as pl
from jax.experimental.pallas import tpu as pltpu
from jax.experimental.pallas import tpu_sc as plsc

BATCH = 16384
D = 64
L = 16
NC, NS = 2, 16
NW = NC * NS
BPW = BATCH // NW
CHUNK = 128
NCHUNK = BPW // CHUNK


def _body(uidx_hbm, iidx_hbm, uT_hbm, iT_hbm, ub_hbm, ib_hbm, out_hbm,
          uidx_v, iidx_v, ucols_v, icols_v, ub_v, ib_v, out_v,
          sem_u, sem_i, sem_b):
    wid = lax.axis_index("s") * NC + lax.axis_index("c")
    base = wid * BPW

    pltpu.sync_copy(uidx_hbm.at[pl.ds(wid * NCHUNK, NCHUNK)], uidx_v)
    pltpu.sync_copy(iidx_hbm.at[pl.ds(wid * NCHUNK, NCHUNK)], iidx_v)

    bias_cps = []
    for c in range(NCHUNK):
        sl = pl.ds(c * CHUNK, CHUNK)
        bias_cps.append(pltpu.async_copy(ub_hbm.at[uidx_v.at[c]],
                                         ub_v.at[sl], sem_b))
        bias_cps.append(pltpu.async_copy(ib_hbm.at[iidx_v.at[c]],
                                         ib_v.at[sl], sem_b))

    def fire(d, carry):
        for c in range(NCHUNK):
            sl = pl.ds(c * CHUNK, CHUNK)
            pltpu.async_copy(uT_hbm.at[d].at[uidx_v.at[c]],
                             ucols_v.at[d, sl], sem_u)
            pltpu.async_copy(iT_hbm.at[d].at[iidx_v.at[c]],
                             icols_v.at[d, sl], sem_i)
        return carry
    lax.fori_loop(0, D, fire, 0)

    pltpu.make_async_copy(uT_hbm.at[pl.ds(0, D), pl.ds(0, BPW)], ucols_v,
                          sem_u).wait()
    pltpu.make_async_copy(iT_hbm.at[pl.ds(0, D), pl.ds(0, BPW)], icols_v,
                          sem_i).wait()
    for cp in bias_cps:
        cp.wait()

    def pos_body(p, carry):
        p0 = pl.multiple_of(p * L, L)
        acc = ub_v[pl.ds(p0, L)] + ib_v[pl.ds(p0, L)]
        for d in range(D):
            acc = acc + ucols_v[d, pl.ds(p0, L)] * icols_v[d, pl.ds(p0, L)]
        out_v[pl.ds(p0, L)] = 4.0 / (1.0 + jnp.exp(-acc)) + 1.0
        return carry
    lax.fori_loop(0, BPW // L, pos_body, 0)

    pltpu.sync_copy(out_v, out_hbm.at[pl.ds(base, BPW)])


@jax.jit
def kernel(user_indices, item_indices, user_emb, item_emb, user_bias,
         item_bias):
    uidx = user_indices.astype(jnp.int32).reshape(NW * NCHUNK, CHUNK)
    iidx = item_indices.astype(jnp.int32).reshape(NW * NCHUNK, CHUNK)
    uT = jnp.pad(user_emb, ((0, 7), (0, 0))).T
    iT = jnp.pad(item_emb, ((0, 7), (0, 0))).T
    ub = user_bias.reshape(-1)
    ib = item_bias.reshape(-1)
    mesh = plsc.VectorSubcoreMesh(core_axis_name="c", subcore_axis_name="s")
    run = pl.kernel(
        _body,
        out_type=jax.ShapeDtypeStruct((BATCH,), jnp.float32),
        mesh=mesh,
        compiler_params=pltpu.CompilerParams(
            use_tc_tiling_on_sc=False,
            needs_layout_passes=False,
        ),
        scratch_types=[
            pltpu.VMEM((NCHUNK, CHUNK), jnp.int32),
            pltpu.VMEM((NCHUNK, CHUNK), jnp.int32),
            pltpu.VMEM((D, BPW), jnp.float32),
            pltpu.VMEM((D, BPW), jnp.float32),
            pltpu.VMEM((BPW,), jnp.float32),
            pltpu.VMEM((BPW,), jnp.float32),
            pltpu.VMEM((BPW,), jnp.float32),
            pltpu.SemaphoreType.DMA,
            pltpu.SemaphoreType.DMA,
            pltpu.SemaphoreType.DMA,
        ],
    )
    return run(uidx, iidx, uT, iT, ub, ib)


# speedup vs baseline: 22.4339x; 22.4339x over previous
import jax
import jax.numpy as jnp
from jax import lax
from jax.experimental import layout as jex_layout
from jax.experimental import pallas as pl
from jax.experimental.pallas import tpu as pltpu
from jax.experimental.pallas import tpu_sc as plsc

BATCH = 16384
D = 64
L = 16
NC, NS = 2, 16
NW = NC * NS
BPW = BATCH // NW
CHUNK = 128
NCHUNK = BPW // CHUNK


def _bias_body(uidx_hbm, iidx_hbm, ub_hbm, ib_hbm, out_hbm,
               uidx_v, iidx_v, bsum_v, tmp_v, sem):
    wid = lax.axis_index("s") * NC + lax.axis_index("c")
    pltpu.sync_copy(uidx_hbm.at[pl.ds(wid * NCHUNK, NCHUNK)], uidx_v)
    pltpu.sync_copy(iidx_hbm.at[pl.ds(wid * NCHUNK, NCHUNK)], iidx_v)
    cps = []
    for c in range(NCHUNK):
        sl = pl.ds(c * CHUNK, CHUNK)
        cps.append(pltpu.async_copy(ub_hbm.at[uidx_v.at[c]], bsum_v.at[sl],
                                    sem))
        cps.append(pltpu.async_copy(ib_hbm.at[iidx_v.at[c]], tmp_v.at[sl],
                                    sem))
    for cp in cps:
        cp.wait()

    def body(c, carry):
        c0 = pl.multiple_of(c * L, L)
        bsum_v[pl.ds(c0, L)] = bsum_v[pl.ds(c0, L)] + tmp_v[pl.ds(c0, L)]
        return carry
    lax.fori_loop(0, BPW // L, body, 0)
    pltpu.sync_copy(bsum_v, out_hbm.at[pl.ds(wid * BPW, BPW)])


def _dot_body(uidx_hbm, iidx_hbm, uemb_hbm, iemb_hbm, bsum_hbm, out_hbm,
              uidx_v, iidx_v, urows_v, irows_v, bsum_v, out_v,
              sem_u0, sem_u1, sem_i0, sem_i1, sem_b):
    wid = lax.axis_index("s") * NC + lax.axis_index("c")
    base = wid * BPW
    sem_u = (sem_u0, sem_u1)
    sem_i = (sem_i0, sem_i1)

    cb = pltpu.async_copy(bsum_hbm.at[pl.ds(base, BPW)], bsum_v, sem_b)
    pltpu.sync_copy(uidx_hbm.at[pl.ds(wid, 1)], uidx_v)
    pltpu.sync_copy(iidx_hbm.at[pl.ds(wid, 1)], iidx_v)

    def fire_chunk(c):
        def fire(blk, carry):
            b0 = pl.multiple_of(blk * L, L)
            ivu = uidx_v[0, pl.ds(c * CHUNK + b0, L)]
            ivi = iidx_v[0, pl.ds(c * CHUNK + b0, L)]
            for j in range(L):
                pltpu.async_copy(uemb_hbm.at[ivu[j]],
                                 urows_v.at[c % 2, b0 + j], sem_u[c % 2])
                pltpu.async_copy(iemb_hbm.at[ivi[j]],
                                 irows_v.at[c % 2, b0 + j], sem_i[c % 2])
            return carry
        lax.fori_loop(0, CHUNK // L, fire, 0)

    def drain_chunk(c):
        pltpu.make_async_copy(uemb_hbm.at[pl.ds(0, CHUNK)],
                              urows_v.at[c % 2], sem_u[c % 2]).wait()
        pltpu.make_async_copy(iemb_hbm.at[pl.ds(0, CHUNK)],
                              irows_v.at[c % 2], sem_i[c % 2]).wait()

    def blk_body_in(buf, coff, blk, carry):
        lane = lax.iota(jnp.int32, L)
        m8 = lane < 8
        fifteen = jnp.full((L,), L - 1, jnp.int32)
        stage_idx = []
        for m in (16, 8, 4, 2):
            h = m // 2
            ia = (lane // h) * m + (lane % h)
            ib = ((lane - 8) // h) * m + ((lane - 8) % h)
            stage_idx.append((lax.bitwise_and(ia, fifteen),
                              lax.bitwise_and(ia + h, fifteen),
                              lax.bitwise_and(ib, fifteen),
                              lax.bitwise_and(ib + h, fifteen)))

        def combine(a, b, st):
            iax, iay, ibx, iby = stage_idx[st]
            x = jnp.where(m8, a[iax], b[ibx])
            y = jnp.where(m8, a[iay], b[iby])
            return x + y

        b0 = pl.multiple_of(blk * L, L)
        ps = []
        for j in range(L):
            b = b0 + j
            p = urows_v[buf, b, pl.ds(0, L)] * irows_v[buf, b, pl.ds(0, L)]
            for cc in range(1, 4):
                p = p + (urows_v[buf, b, pl.ds(cc * L, L)]
                         * irows_v[buf, b, pl.ds(cc * L, L)])
            ps.append(p)
        for st in range(4):
            ps = [combine(ps[2 * k], ps[2 * k + 1], st)
                  for k in range(len(ps) // 2)]
        g0 = coff + blk * L
        x = ps[0] + bsum_v[pl.ds(g0, L)]
        out_v[pl.ds(g0, L)] = 4.0 / (1.0 + jnp.exp(-x)) + 1.0
        return carry

    fire_chunk(0)
    for c in range(NCHUNK):
        if c + 1 < NCHUNK:
            fire_chunk(c + 1)
        drain_chunk(c)
        if c == 0:
            cb.wait()

        def blk_body(blk, carry, _buf=c % 2, _coff=c * CHUNK):
            return blk_body_in(_buf, _coff, blk, carry)

        lax.fori_loop(0, CHUNK // L, blk_body, 0)
    pltpu.sync_copy(out_v, out_hbm.at[pl.ds(base, BPW)])


@jax.jit
def kernel(user_indices, item_indices, user_emb, item_emb, user_bias,
           item_bias):
    mesh = plsc.VectorSubcoreMesh(core_axis_name="c", subcore_axis_name="s")
    uidx_c = user_indices.astype(jnp.int32).reshape(NW * NCHUNK, CHUNK)
    iidx_c = item_indices.astype(jnp.int32).reshape(NW * NCHUNK, CHUNK)
    bias_run = pl.kernel(
        _bias_body,
        out_type=jax.ShapeDtypeStruct((BATCH,), jnp.float32),
        mesh=mesh,
        compiler_params=pltpu.CompilerParams(
            use_tc_tiling_on_sc=False,
            needs_layout_passes=False,
        ),
        scratch_types=[
            pltpu.VMEM((NCHUNK, CHUNK), jnp.int32),
            pltpu.VMEM((NCHUNK, CHUNK), jnp.int32),
            pltpu.VMEM((BPW,), jnp.float32),
            pltpu.VMEM((BPW,), jnp.float32),
            pltpu.SemaphoreType.DMA,
        ],
    )
    bsum = bias_run(uidx_c, iidx_c, user_bias.reshape(-1),
                    item_bias.reshape(-1))

    uidx_r = user_indices.astype(jnp.int32).reshape(NW, BPW)
    iidx_r = item_indices.astype(jnp.int32).reshape(NW, BPW)

    row_major = jex_layout.Layout(major_to_minor=(0, 1), tiling=((8, 128),))
    uemb_c = jex_layout.with_layout_constraint(user_emb, row_major)
    iemb_c = jex_layout.with_layout_constraint(item_emb, row_major)
    uemb_c, iemb_c = lax.optimization_barrier((uemb_c, iemb_c))
    dot_run = pl.kernel(
        _dot_body,
        out_type=jax.ShapeDtypeStruct((BATCH,), jnp.float32),
        mesh=mesh,
        compiler_params=pltpu.CompilerParams(
            use_tc_tiling_on_sc=True,
            needs_layout_passes=False,
        ),
        scratch_types=[
            pltpu.VMEM((1, BPW), jnp.int32),
            pltpu.VMEM((1, BPW), jnp.int32),
            pltpu.VMEM((2, CHUNK, D), jnp.float32),
            pltpu.VMEM((2, CHUNK, D), jnp.float32),
            pltpu.VMEM((BPW,), jnp.float32),
            pltpu.VMEM((BPW,), jnp.float32),
            pltpu.SemaphoreType.DMA,
            pltpu.SemaphoreType.DMA,
            pltpu.SemaphoreType.DMA,
            pltpu.SemaphoreType.DMA,
            pltpu.SemaphoreType.DMA,
        ],
    )
    return dot_run(uidx_r, iidx_r, uemb_c, iemb_c, bsum)
